# baseline (speedup 1.0000x reference)
import functools

import jax
import jax.numpy as jnp
from jax import lax
from jax.experimental import pallas as pl
from jax.experimental.pallas import tpu as pltpu
from jax.experimental.pallas import tpu_sc as plsc

_CUT = (0, 20000, 60000, 100000)
_DIMS = (1024, 256, 64)
_SIZES = (20000, 40000, 40000)
_OUT_DIM = 1024

_NC, _NS = 2, 16
_NW = _NC * _NS
_CHUNK = (32, 64, 64)


_G = 64
_NPAD = 256


def _lane_bcast(vec, idx16):
    dn = lax.GatherDimensionNumbers(offset_dims=(), collapsed_slice_dims=(0,),
                                    start_index_map=(0,))
    return lax.gather(vec, idx16[:, None], dn, slice_sizes=(1,),
                      mode=lax.GatherScatterMode.PROMISE_IN_BOUNDS)


def _prefix_sum16(x, iota16):
    z16 = iota16 * 0
    y = x
    for k in (1, 2, 4, 8):
        shifted = _lane_bcast(y, jnp.maximum(iota16 - k, 0))
        y = y + jnp.where(iota16 >= k, shifted, z16)
    return y


def _pack_bf16_rows(s, R):
    def row_body(r, carry):
        for k in range(32):
            va = s[r, pl.ds(k * 32, 16)]
            vb = s[r, pl.ds(k * 32 + 16, 16)]
            lo = lax.shift_right_logical(va + 0x8000, 16)
            hi = (vb + 0x8000) & (-65536)
            s[r, pl.ds(k * 16, 16)] = hi | lo
        return carry
    lax.fori_loop(0, R, row_body, 0)


def _sc_gather_body(x_hbm, e0_hbm, e1_hbm, e2_hbm,
                    c0_hbm, c1_hbm, c2_hbm,
                    x_v, idx0_v, idx1_v, idx2_v,
                    s0a, s0b, s1a, s1b, s2a, s2b, sema, semb, T):
    wid = lax.axis_index("s") * _NC + lax.axis_index("c")
    base = wid * T
    pltpu.sync_copy(x_hbm.at[pl.ds(base, T)], x_v)

    iota16 = lax.iota(jnp.int32, 16)
    for i in range(T // 16):
        v = x_v[pl.ds(i * 16, 16)]
        tok = base + i * 16 + iota16
        m0 = v < _CUT[1]
        m1 = jnp.logical_and(v >= _CUT[1], v < _CUT[2])
        m2 = v >= _CUT[2]
        idx0_v[pl.ds(i * 16, 16)] = jnp.where(m0, v, tok)
        idx1_v[pl.ds(i * 16, 16)] = jnp.where(m1, v - _CUT[1], tok)
        idx2_v[pl.ds(i * 16, 16)] = jnp.where(m2, (v - _CUT[2]) >> 1, tok)

    emb = (e0_hbm, e1_hbm, e2_hbm)
    cbuf = (c0_hbm, c1_hbm, c2_hbm)
    idx = (idx0_v, idx1_v, idx2_v)
    slabs = ((s0a, s0b), (s1a, s1b), (s2a, s2b))
    sems = (sema, semb)
    for t in range(3):
        R = _CHUNK[t]
        nch = T // R
        d = pltpu.async_copy(emb[t].at[idx[t].at[pl.ds(0, R)]],
                             slabs[t][0], sems[0])
        for c in range(nch):
            d.wait()
            if c + 1 < nch:
                d = pltpu.async_copy(
                    emb[t].at[idx[t].at[pl.ds((c + 1) * R, R)]],
                    slabs[t][(c + 1) & 1], sems[(c + 1) & 1])
            s = slabs[t][c & 1]
            if t == 0:
                _pack_bf16_rows(s, R)
                src = s.at[:, pl.ds(0, _DIMS[0] // 2)]
            else:
                src = s
            pltpu.sync_copy(src, cbuf[t].at[pl.ds(base + c * R, R)])


def _sc_gather(x_flat, e0, e1, e2):
    n = x_flat.shape[0]
    T = n // _NW
    mesh = plsc.VectorSubcoreMesh(core_axis_name="c", subcore_axis_name="s",
                                  num_cores=_NC, num_subcores=_NS)
    body = functools.partial(_sc_gather_body, T=T)
    return pl.kernel(
        body,
        out_type=(
            jax.ShapeDtypeStruct((n, _DIMS[0] // 2), jnp.int32),
            jax.ShapeDtypeStruct((n, _DIMS[1]), jnp.float32),
            jax.ShapeDtypeStruct((n, 2 * _DIMS[2]), jnp.float32),
        ),
        mesh=mesh,
        scratch_types=[
            pltpu.VMEM((T,), jnp.int32),
            pltpu.VMEM((T,), jnp.int32),
            pltpu.VMEM((T,), jnp.int32),
            pltpu.VMEM((T,), jnp.int32),
            pltpu.VMEM((_CHUNK[0], _DIMS[0]), jnp.int32),
            pltpu.VMEM((_CHUNK[0], _DIMS[0]), jnp.int32),
            pltpu.VMEM((_CHUNK[1], _DIMS[1]), jnp.float32),
            pltpu.VMEM((_CHUNK[1], _DIMS[1]), jnp.float32),
            pltpu.VMEM((_CHUNK[2], 2 * _DIMS[2]), jnp.float32),
            pltpu.VMEM((_CHUNK[2], 2 * _DIMS[2]), jnp.float32),
            pltpu.SemaphoreType.DMA,
            pltpu.SemaphoreType.DMA,
        ],
        name="adaptive_input_sc_gather",
    )(x_flat, e0, e1, e2)


def _tc_project_body(x_ref, c0_ref, c1_ref, c2_ref, w0l_ref, w0h_ref,
                     w1_ref, w2_ref, o_ref):
    xb = x_ref[...]
    m0 = xb < _CUT[1]
    m1 = jnp.logical_and(xb >= _CUT[1], xb < _CUT[2])
    m2 = xb >= _CUT[2]
    val = c0_ref[...]
    lo = lax.bitcast_convert_type(lax.shift_left(val, 16), jnp.float32)
    hi = lax.bitcast_convert_type(val & (-65536), jnp.float32)
    a0l = jnp.where(m0, lo, 0.0).astype(jnp.bfloat16)
    a0h = jnp.where(m0, hi, 0.0).astype(jnp.bfloat16)
    a1 = jnp.where(m1, c1_ref[...], 0.0)
    even = jnp.equal(jnp.bitwise_and(xb, 1), 0)
    half = jnp.where(even, c2_ref[:, : _DIMS[2]], c2_ref[:, _DIMS[2]:])
    a2 = jnp.where(m2, half, 0.0)
    dn = (((1,), (1,)), ((), ()))
    acc = lax.dot_general(a0l, w0l_ref[...], dn,
                          preferred_element_type=jnp.float32)
    acc = acc + lax.dot_general(a0h, w0h_ref[...], dn,
                                preferred_element_type=jnp.float32)
    acc = acc + lax.dot_general(a1, w1_ref[...], dn,
                                preferred_element_type=jnp.float32)
    acc = acc + lax.dot_general(a2, w2_ref[...], dn,
                                preferred_element_type=jnp.float32)
    o_ref[...] = acc


def _tc_project(x_col, c0, c1, c2, w0l, w0h, w1, w2):
    n = x_col.shape[0]
    B = 2048
    grid = (n // B,)
    return pl.pallas_call(
        _tc_project_body,
        grid=grid,
        in_specs=[
            pl.BlockSpec((B, 1), lambda i: (i, 0)),
            pl.BlockSpec((B, _DIMS[0] // 2), lambda i: (i, 0)),
            pl.BlockSpec((B, _DIMS[1]), lambda i: (i, 0)),
            pl.BlockSpec((B, 2 * _DIMS[2]), lambda i: (i, 0)),
            pl.BlockSpec((_OUT_DIM, _DIMS[0] // 2), lambda i: (0, 0)),
            pl.BlockSpec((_OUT_DIM, _DIMS[0] // 2), lambda i: (0, 0)),
            pl.BlockSpec((_OUT_DIM, _DIMS[1]), lambda i: (0, 0)),
            pl.BlockSpec((_OUT_DIM, _DIMS[2]), lambda i: (0, 0)),
        ],
        out_specs=pl.BlockSpec((B, _OUT_DIM), lambda i: (i, 0)),
        out_shape=jax.ShapeDtypeStruct((n, _OUT_DIM), jnp.float32),
        name="adaptive_input_tc_project",
    )(x_col, c0, c1, c2, w0l, w0h, w1, w2)


def kernel(x, emb0, emb1, emb2, W0, W1, W2):
    orig_shape = x.shape
    x_flat = x.reshape(-1).astype(jnp.int32)
    e2_pairs = emb2.reshape(emb2.shape[0] // 2, 2 * _DIMS[2])
    e0_bits = lax.bitcast_convert_type(emb0, jnp.int32)
    c0, c1, c2 = _sc_gather(x_flat, e0_bits, emb1, e2_pairs)
    w0v = W0.reshape(_OUT_DIM, 32, 32)
    w0l = w0v[:, :, :16].reshape(_OUT_DIM, _DIMS[0] // 2).astype(jnp.bfloat16)
    w0h = w0v[:, :, 16:].reshape(_OUT_DIM, _DIMS[0] // 2).astype(jnp.bfloat16)
    out = _tc_project(x_flat.reshape(-1, 1), c0, c1, c2, w0l, w0h, W1, W2)
    return out.reshape(orig_shape + (_OUT_DIM,))


# speedup vs baseline: 1.2164x; 1.2164x over previous
import functools

import jax
import jax.numpy as jnp
from jax import lax
from jax.experimental import pallas as pl
from jax.experimental.pallas import tpu as pltpu
from jax.experimental.pallas import tpu_sc as plsc

_CUT = (0, 20000, 60000, 100000)
_DIMS = (1024, 256, 64)
_SIZES = (20000, 40000, 40000)
_OUT_DIM = 1024

_NC, _NS = 2, 16
_NW = _NC * _NS
_CHUNK = (32, 64, 64)


_G = 64
_NPAD = 256


def _lane_bcast(vec, idx16):
    dn = lax.GatherDimensionNumbers(offset_dims=(), collapsed_slice_dims=(0,),
                                    start_index_map=(0,))
    return lax.gather(vec, idx16[:, None], dn, slice_sizes=(1,),
                      mode=lax.GatherScatterMode.PROMISE_IN_BOUNDS)


def _prefix_sum16(x, iota16):
    z16 = iota16 * 0
    y = x
    for k in (1, 2, 4, 8):
        shifted = _lane_bcast(y, jnp.maximum(iota16 - k, 0))
        y = y + jnp.where(iota16 >= k, shifted, z16)
    return y


def _pack_bf16_rows(s, R):
    def row_body(r, carry):
        for k in range(32):
            va = s[r, pl.ds(k * 32, 16)]
            vb = s[r, pl.ds(k * 32 + 16, 16)]
            lo = lax.shift_right_logical(va + 0x8000, 16)
            hi = (vb + 0x8000) & (-65536)
            s[r, pl.ds(k * 16, 16)] = hi | lo
        return carry
    lax.fori_loop(0, R, row_body, 0)


def _sc_gather_body(x_hbm, e0_hbm, e1_hbm, e2_hbm,
                    c0_hbm, c1_hbm, c2_hbm,
                    x_v, idx0_v, idx1_v, idx2_v,
                    s0a, s0b, s1a, s1b, s2a, s2b, sema, semb, T):
    wid = lax.axis_index("s") * _NC + lax.axis_index("c")
    base = wid * T
    pltpu.sync_copy(x_hbm.at[pl.ds(base, T)], x_v)

    iota16 = lax.iota(jnp.int32, 16)
    for i in range(T // 16):
        v = x_v[pl.ds(i * 16, 16)]
        tok = base + i * 16 + iota16
        m0 = v < _CUT[1]
        m1 = jnp.logical_and(v >= _CUT[1], v < _CUT[2])
        m2 = v >= _CUT[2]
        idx0_v[pl.ds(i * 16, 16)] = jnp.where(m0, v, tok)
        idx1_v[pl.ds(i * 16, 16)] = jnp.where(m1, v - _CUT[1], tok)
        idx2_v[pl.ds(i * 16, 16)] = jnp.where(m2, (v - _CUT[2]) >> 1, tok)

    emb = (e0_hbm, e1_hbm, e2_hbm)
    cbuf = (c0_hbm, c1_hbm, c2_hbm)
    idx = (idx0_v, idx1_v, idx2_v)
    slabs = ((s0a, s0b), (s1a, s1b), (s2a, s2b))
    sems = (sema, semb)
    for t in range(3):
        R = _CHUNK[t]
        nch = T // R
        d = pltpu.async_copy(emb[t].at[idx[t].at[pl.ds(0, R)]],
                             slabs[t][0], sems[0])
        for c in range(nch):
            d.wait()
            if c + 1 < nch:
                d = pltpu.async_copy(
                    emb[t].at[idx[t].at[pl.ds((c + 1) * R, R)]],
                    slabs[t][(c + 1) & 1], sems[(c + 1) & 1])
            pltpu.sync_copy(slabs[t][c & 1],
                            cbuf[t].at[pl.ds(base + c * R, R)])


def _sc_gather(x_flat, e0, e1, e2):
    n = x_flat.shape[0]
    T = n // _NW
    mesh = plsc.VectorSubcoreMesh(core_axis_name="c", subcore_axis_name="s",
                                  num_cores=_NC, num_subcores=_NS)
    body = functools.partial(_sc_gather_body, T=T)
    return pl.kernel(
        body,
        out_type=(
            jax.ShapeDtypeStruct((n, _DIMS[0]), jnp.float32),
            jax.ShapeDtypeStruct((n, _DIMS[1]), jnp.float32),
            jax.ShapeDtypeStruct((n, 2 * _DIMS[2]), jnp.float32),
        ),
        mesh=mesh,
        scratch_types=[
            pltpu.VMEM((T,), jnp.int32),
            pltpu.VMEM((T,), jnp.int32),
            pltpu.VMEM((T,), jnp.int32),
            pltpu.VMEM((T,), jnp.int32),
            pltpu.VMEM((_CHUNK[0], _DIMS[0]), jnp.float32),
            pltpu.VMEM((_CHUNK[0], _DIMS[0]), jnp.float32),
            pltpu.VMEM((_CHUNK[1], _DIMS[1]), jnp.float32),
            pltpu.VMEM((_CHUNK[1], _DIMS[1]), jnp.float32),
            pltpu.VMEM((_CHUNK[2], 2 * _DIMS[2]), jnp.float32),
            pltpu.VMEM((_CHUNK[2], 2 * _DIMS[2]), jnp.float32),
            pltpu.SemaphoreType.DMA,
            pltpu.SemaphoreType.DMA,
        ],
        name="adaptive_input_sc_gather",
    )(x_flat, e0, e1, e2)


def _tc_project_body(x_ref, c0_ref, c1_ref, c2_ref, w0_ref,
                     w1_ref, w2_ref, o_ref):
    xb = x_ref[...]
    m0 = xb < _CUT[1]
    m1 = jnp.logical_and(xb >= _CUT[1], xb < _CUT[2])
    m2 = xb >= _CUT[2]
    a0 = jnp.where(m0, c0_ref[...], 0.0).astype(jnp.bfloat16)
    a1 = jnp.where(m1, c1_ref[...], 0.0).astype(jnp.bfloat16)
    even = jnp.equal(jnp.bitwise_and(xb, 1), 0)
    half = jnp.where(even, c2_ref[:, : _DIMS[2]], c2_ref[:, _DIMS[2]:])
    a2 = jnp.where(m2, half, 0.0).astype(jnp.bfloat16)
    dn = (((1,), (1,)), ((), ()))
    acc = lax.dot_general(a0, w0_ref[...], dn,
                          preferred_element_type=jnp.float32)
    acc = acc + lax.dot_general(a1, w1_ref[...], dn,
                                preferred_element_type=jnp.float32)
    acc = acc + lax.dot_general(a2, w2_ref[...], dn,
                                preferred_element_type=jnp.float32)
    o_ref[...] = acc


def _tc_project(x_col, c0, c1, c2, w0, w1, w2):
    n = x_col.shape[0]
    B = 2048
    grid = (n // B,)
    return pl.pallas_call(
        _tc_project_body,
        grid=grid,
        in_specs=[
            pl.BlockSpec((B, 1), lambda i: (i, 0)),
            pl.BlockSpec((B, _DIMS[0]), lambda i: (i, 0)),
            pl.BlockSpec((B, _DIMS[1]), lambda i: (i, 0)),
            pl.BlockSpec((B, 2 * _DIMS[2]), lambda i: (i, 0)),
            pl.BlockSpec((_OUT_DIM, _DIMS[0]), lambda i: (0, 0)),
            pl.BlockSpec((_OUT_DIM, _DIMS[1]), lambda i: (0, 0)),
            pl.BlockSpec((_OUT_DIM, _DIMS[2]), lambda i: (0, 0)),
        ],
        out_specs=pl.BlockSpec((B, _OUT_DIM), lambda i: (i, 0)),
        out_shape=jax.ShapeDtypeStruct((n, _OUT_DIM), jnp.float32),
        name="adaptive_input_tc_project",
    )(x_col, c0, c1, c2, w0, w1, w2)


def kernel(x, emb0, emb1, emb2, W0, W1, W2):
    orig_shape = x.shape
    x_flat = x.reshape(-1).astype(jnp.int32)
    e2_pairs = emb2.reshape(emb2.shape[0] // 2, 2 * _DIMS[2])
    c0, c1, c2 = _sc_gather(x_flat, emb0, emb1, e2_pairs)
    out = _tc_project(x_flat.reshape(-1, 1), c0, c1, c2,
                      W0.astype(jnp.bfloat16), W1.astype(jnp.bfloat16),
                      W2.astype(jnp.bfloat16))
    return out.reshape(orig_shape + (_OUT_DIM,))
